# baseline (speedup 1.0000x reference)
import jax
import jax.numpy as jnp
from jax import lax
from jax.experimental import pallas as pl
from jax.experimental.pallas import tpu as pltpu
from jax.experimental.pallas import tpu_sc as plsc

B = 16384
A = 1000
NC = 2
NS = 16
RPW = 40
NWORK = A // RPW
CL = 1024
NCH = B // CL
NBUF = 2


def _sc_body(sav_hbm, act_hbm, q_hbm, out_hbm, act_v, q_v, bufs, lsems, ssems):
    wid = lax.axis_index("s") * NC + lax.axis_index("c")
    r0 = wid * RPW

    @pl.when(wid < NWORK)
    def _():
        def start_load(g):
            b = g % NBUF
            return pltpu.async_copy(
                sav_hbm.at[pl.ds(r0, RPW), pl.ds(g * CL, CL)], bufs.at[b],
                lsems[b])

        def start_store(g):
            b = g % NBUF
            return pltpu.async_copy(
                bufs.at[b], out_hbm.at[pl.ds(r0, RPW), pl.ds(g * CL, CL)],
                ssems[b])

        loads = {}
        stores = {}
        for g in range(min(NBUF - 1, NCH)):
            loads[g] = start_load(g)

        lane = lax.iota(jnp.int32, 16)
        for g in range(NCH):
            b = g % NBUF
            pltpu.sync_copy(act_hbm.at[pl.ds(g * CL, CL)], act_v)
            pltpu.sync_copy(q_hbm.at[pl.ds(g * CL, CL)], q_v)
            loads.pop(g).wait()
            for k in range(CL // 16):
                cols = act_v[pl.ds(k * 16, 16)]
                vals = q_v[pl.ds(k * 16, 16)]
                mask = (cols >= r0) & (cols < r0 + RPW)
                plsc.store_scatter(
                    bufs.at[b], [cols - r0, lane + k * 16], vals, mask=mask)
            stores[g] = start_store(g)
            nxt = g + NBUF - 1
            if nxt < NCH:
                if nxt >= NBUF:
                    stores.pop(nxt - NBUF).wait()
                loads[nxt] = start_load(nxt)
        for g in sorted(stores):
            stores[g].wait()


def kernel(state_action_values, action, q_prime):
    act = action[:, 0].astype(jnp.int32)
    sav_t = state_action_values.T
    mesh = plsc.VectorSubcoreMesh(
        core_axis_name="c", subcore_axis_name="s", num_cores=NC,
        num_subcores=NS)
    sc_call = pl.kernel(
        _sc_body,
        out_type=jax.ShapeDtypeStruct((A, B), jnp.float32),
        mesh=mesh,
        compiler_params=pltpu.CompilerParams(needs_layout_passes=False),
        scratch_types=[
            pltpu.VMEM((CL,), jnp.int32),
            pltpu.VMEM((CL,), jnp.float32),
            pltpu.VMEM((NBUF, RPW, CL), jnp.float32),
            [pltpu.SemaphoreType.DMA] * NBUF,
            [pltpu.SemaphoreType.DMA] * NBUF,
        ],
    )
    return sc_call(sav_t, act, q_prime).T


# speedup vs baseline: 2.3616x; 2.3616x over previous
import jax
import jax.numpy as jnp
from jax.experimental import pallas as pl
from jax.experimental.pallas import tpu as pltpu

B = 16384
A = 1000
BL = 2048


def _scatter_copy_kernel(act_ref, q_ref, sav_ref, out_ref):
    act = act_ref[:]
    q = q_ref[:]
    col = jax.lax.broadcasted_iota(jnp.int32, (A, BL), 0)
    mask = col == act[None, :]
    out_ref[...] = jnp.where(mask, q[None, :], sav_ref[...])


def kernel(state_action_values, action, q_prime):
    act = action[:, 0].astype(jnp.int32)
    sav_t = state_action_values.T
    grid = (B // BL,)
    out_t = pl.pallas_call(
        _scatter_copy_kernel,
        grid=grid,
        in_specs=[
            pl.BlockSpec((BL,), lambda i: (i,)),
            pl.BlockSpec((BL,), lambda i: (i,)),
            pl.BlockSpec((A, BL), lambda i: (0, i)),
        ],
        out_specs=pl.BlockSpec((A, BL), lambda i: (0, i)),
        out_shape=jax.ShapeDtypeStruct((A, B), jnp.float32),
        compiler_params=pltpu.CompilerParams(
            vmem_limit_bytes=100 * 1024 * 1024),
    )(act, q_prime, sav_t)
    return out_t.T
